# Initial kernel scaffold; baseline (speedup 1.0000x reference)
import jax
import jax.numpy as jnp
from jax.experimental import pallas as pl


def kernel(x, gate_W, gate_b, w1, b1, w2, b2, w3, b3):
    raise NotImplementedError("write your pallas kernel here")


# speedup vs baseline: 2.6279x; 2.6279x over previous
import functools

import jax
import jax.numpy as jnp
from jax.experimental import pallas as pl

D_MODEL = 768
D_FF = 2048
N_EXPERTS = 8
N_TOKENS = 2048
BT = 256
LANES = 128


def _gate_kernel(x_ref, gw_ref, gb_ref, w_ref):
    l = jnp.dot(x_ref[:], gw_ref[:], preferred_element_type=jnp.float32)
    l = l + gb_ref[:]
    col = jax.lax.broadcasted_iota(jnp.int32, l.shape, 1)
    neg = jnp.float32(-1e30)
    l = jnp.where(col < N_EXPERTS, l, neg)
    m1 = jnp.max(l, axis=1, keepdims=True)
    i1 = jnp.min(jnp.where(l >= m1, col, LANES), axis=1, keepdims=True)
    s = jnp.sum(jnp.exp(l - m1), axis=1, keepdims=True)
    l2 = jnp.where(col == i1, neg, l)
    m2 = jnp.max(l2, axis=1, keepdims=True)
    i2 = jnp.min(jnp.where(l2 >= m2, col, LANES), axis=1, keepdims=True)
    p1 = 1.0 / s
    p2 = jnp.exp(m2 - m1) / s
    d = p1 + p2 + 1e-6
    p1n = p1 / d
    p2n = p2 / d
    w = jnp.where(col == i1, p1n, 0.0) + jnp.where(col == i2, p2n, 0.0)
    w_ref[:] = w


def _ff_kernel(x_ref, s_ref, w1_ref, b1_ref, w3_ref, b3_ref, w2_ref, b2_ref,
               out_ref):
    e = pl.program_id(0)
    tb = pl.program_id(1)
    xb = x_ref[:]
    h1 = jnp.dot(xb, w1_ref[0], preferred_element_type=jnp.float32) + b1_ref[0]
    h3 = jnp.dot(xb, w3_ref[0], preferred_element_type=jnp.float32) + b3_ref[0]
    h = h1 * jax.nn.sigmoid(h1) * h3
    y = jnp.dot(h, w2_ref[0], preferred_element_type=jnp.float32)
    sblk = s_ref[:]
    col = jax.lax.broadcasted_iota(jnp.int32, sblk.shape, 1)
    sel = jnp.sum(jnp.where(col == e, sblk, 0.0), axis=1, keepdims=True)
    y = (y + b2_ref[0]) * sel
    rows = pl.ds(tb * BT, BT)

    @pl.when(e == 0)
    def _():
        out_ref[rows, :] = y

    @pl.when(e != 0)
    def _():
        out_ref[rows, :] += y


@functools.partial(jax.jit, static_argnames=())
def kernel(x, gate_W, gate_b, w1, b1, w2, b2, w3, b3):
    gwp = jnp.pad(gate_W, ((0, 0), (0, LANES - N_EXPERTS)))
    gbp = jnp.pad(gate_b, (0, LANES - N_EXPERTS)).reshape(1, LANES)
    w_te = pl.pallas_call(
        _gate_kernel,
        out_shape=jax.ShapeDtypeStruct((N_TOKENS, LANES), jnp.float32),
    )(x, gwp, gbp)

    b1r = b1.reshape(N_EXPERTS, 1, D_FF)
    b3r = b3.reshape(N_EXPERTS, 1, D_FF)
    b2r = b2.reshape(N_EXPERTS, 1, D_MODEL)
    grid = (N_EXPERTS, N_TOKENS // BT)
    out = pl.pallas_call(
        _ff_kernel,
        grid=grid,
        in_specs=[
            pl.BlockSpec((BT, D_MODEL), lambda e, tb: (tb, 0)),
            pl.BlockSpec((BT, LANES), lambda e, tb: (tb, 0)),
            pl.BlockSpec((1, D_MODEL, D_FF), lambda e, tb: (e, 0, 0)),
            pl.BlockSpec((1, 1, D_FF), lambda e, tb: (e, 0, 0)),
            pl.BlockSpec((1, D_MODEL, D_FF), lambda e, tb: (e, 0, 0)),
            pl.BlockSpec((1, 1, D_FF), lambda e, tb: (e, 0, 0)),
            pl.BlockSpec((1, D_FF, D_MODEL), lambda e, tb: (e, 0, 0)),
            pl.BlockSpec((1, 1, D_MODEL), lambda e, tb: (e, 0, 0)),
        ],
        out_specs=pl.BlockSpec((N_TOKENS, D_MODEL), lambda e, tb: (0, 0)),
        out_shape=jax.ShapeDtypeStruct((N_TOKENS, D_MODEL), jnp.float32),
    )(x, w_te, w1, b1r, w3, b3r, w2, b2r)
    return out


# speedup vs baseline: 3.1115x; 1.1840x over previous
import functools

import jax
import jax.numpy as jnp
from jax.experimental import pallas as pl
from jax.experimental.pallas import tpu as pltpu

D_MODEL = 768
D_FF = 2048
N_EXPERTS = 8
N_TOKENS = 2048
LANES = 128
BR = 256
LOG2_BR = 8
ROWS = N_TOKENS * 2 + N_EXPERTS * BR
NBLK = ROWS // BR


def _gate_kernel(x_ref, gw_ref, gb_ref, w_ref, pos1_ref, pos2_ref,
                 p1_ref, p2_ref, pc_ref):
    l = jnp.dot(x_ref[:], gw_ref[:], preferred_element_type=jnp.float32)
    l = l + gb_ref[:]
    col = jax.lax.broadcasted_iota(jnp.int32, l.shape, 1)
    neg = jnp.float32(-1e30)
    l = jnp.where(col < N_EXPERTS, l, neg)
    m1 = jnp.max(l, axis=1, keepdims=True)
    i1 = jnp.min(jnp.where(l >= m1, col, LANES), axis=1, keepdims=True)
    s = jnp.sum(jnp.exp(l - m1), axis=1, keepdims=True)
    l2 = jnp.where(col == i1, neg, l)
    m2 = jnp.max(l2, axis=1, keepdims=True)
    i2 = jnp.min(jnp.where(l2 >= m2, col, LANES), axis=1, keepdims=True)
    p1 = 1.0 / s
    p2 = jnp.exp(m2 - m1) / s
    d = p1 + p2 + 1e-6
    p1_ref[:] = p1 / d
    p2_ref[:] = p2 / d
    oh1 = jnp.where(col == i1, 1.0, 0.0)
    oh2 = jnp.where(col == i2, 1.0, 0.0)
    w_ref[:] = oh1 * (p1 / d) + oh2 * (p2 / d)
    tot = oh1 + oh2
    r_t = jax.lax.broadcasted_iota(jnp.int32, (N_TOKENS, N_TOKENS), 0)
    c_t = jax.lax.broadcasted_iota(jnp.int32, (N_TOKENS, N_TOKENS), 1)
    lt = jnp.where(r_t > c_t, 1.0, 0.0).astype(jnp.bfloat16)
    excl = jnp.dot(lt, tot.astype(jnp.bfloat16),
                   preferred_element_type=jnp.float32)
    counts = jnp.sum(tot, axis=0, keepdims=True)
    pci = counts.astype(jnp.int32)
    pc = ((pci + (BR - 1)) >> LOG2_BR) << LOG2_BR
    pc_ref[:] = pc
    r_e = jax.lax.broadcasted_iota(jnp.int32, (LANES, LANES), 0)
    c_e = jax.lax.broadcasted_iota(jnp.int32, (LANES, LANES), 1)
    lte = jnp.where(r_e < c_e, 1.0, 0.0)
    seg = jnp.dot(pc.astype(jnp.float32), lte,
                  preferred_element_type=jnp.float32)
    segex = excl + seg
    pos1_ref[:] = jnp.sum(jnp.where(col == i1, segex, 0.0), axis=1,
                          keepdims=True).astype(jnp.int32)
    pos2_ref[:] = jnp.sum(jnp.where(col == i2, segex, 0.0), axis=1,
                          keepdims=True).astype(jnp.int32)


def _ff_sparse_kernel(be_ref, valid_ref, xs_ref, w1_ref, b1_ref, w3_ref,
                      b3_ref, w2_ref, b2_ref, ys_ref):
    g = pl.program_id(0)

    @pl.when(valid_ref[g] == 1)
    def _():
        xb = xs_ref[:]
        h1 = jnp.dot(xb, w1_ref[0], preferred_element_type=jnp.float32)
        h1 = h1 + b1_ref[0]
        h3 = jnp.dot(xb, w3_ref[0], preferred_element_type=jnp.float32)
        h3 = h3 + b3_ref[0]
        h = h1 * jax.nn.sigmoid(h1) * h3
        ys_ref[:] = jnp.dot(h, w2_ref[0],
                            preferred_element_type=jnp.float32) + b2_ref[0]


def _combine_kernel(g1_ref, g2_ref, p1_ref, p2_ref, out_ref):
    out_ref[:] = g1_ref[:] * p1_ref[:] + g2_ref[:] * p2_ref[:]


@functools.partial(jax.jit, static_argnames=())
def kernel(x, gate_W, gate_b, w1, b1, w2, b2, w3, b3):
    f32 = jnp.float32
    gwp = jnp.pad(gate_W, ((0, 0), (0, LANES - N_EXPERTS)))
    gbp = jnp.pad(gate_b, (0, LANES - N_EXPERTS)).reshape(1, LANES)
    w_te, pos1, pos2, p1c, p2c, pc_row = pl.pallas_call(
        _gate_kernel,
        out_shape=(
            jax.ShapeDtypeStruct((N_TOKENS, LANES), f32),
            jax.ShapeDtypeStruct((N_TOKENS, 1), jnp.int32),
            jax.ShapeDtypeStruct((N_TOKENS, 1), jnp.int32),
            jax.ShapeDtypeStruct((N_TOKENS, 1), f32),
            jax.ShapeDtypeStruct((N_TOKENS, 1), f32),
            jax.ShapeDtypeStruct((1, LANES), jnp.int32),
        ),
    )(x, gwp, gbp)

    pc8 = pc_row[0, :N_EXPERTS]
    ends = jnp.cumsum(pc8)
    gbase = jnp.arange(NBLK, dtype=jnp.int32) * BR
    be = jnp.sum((ends[None, :] <= gbase[:, None]).astype(jnp.int32), axis=1)
    valid = (gbase < ends[-1]).astype(jnp.int32)
    max_e = jnp.max(jnp.where(pc8 > 0, jnp.arange(N_EXPERTS, dtype=jnp.int32),
                              0))
    be = jnp.minimum(be, max_e)

    p1f = pos1[:, 0]
    p2f = pos2[:, 0]
    xs = jnp.zeros((ROWS, D_MODEL), f32).at[p1f].set(x).at[p2f].set(x)

    b1r = b1.reshape(N_EXPERTS, 1, D_FF)
    b3r = b3.reshape(N_EXPERTS, 1, D_FF)
    b2r = b2.reshape(N_EXPERTS, 1, D_MODEL)
    ys = pl.pallas_call(
        _ff_sparse_kernel,
        grid_spec=pltpu.PrefetchScalarGridSpec(
            num_scalar_prefetch=2,
            grid=(NBLK,),
            in_specs=[
                pl.BlockSpec((BR, D_MODEL), lambda g, be, va: (g, 0)),
                pl.BlockSpec((1, D_MODEL, D_FF), lambda g, be, va: (be[g], 0, 0)),
                pl.BlockSpec((1, 1, D_FF), lambda g, be, va: (be[g], 0, 0)),
                pl.BlockSpec((1, D_MODEL, D_FF), lambda g, be, va: (be[g], 0, 0)),
                pl.BlockSpec((1, 1, D_FF), lambda g, be, va: (be[g], 0, 0)),
                pl.BlockSpec((1, D_FF, D_MODEL), lambda g, be, va: (be[g], 0, 0)),
                pl.BlockSpec((1, 1, D_MODEL), lambda g, be, va: (be[g], 0, 0)),
            ],
            out_specs=pl.BlockSpec((BR, D_MODEL), lambda g, be, va: (g, 0)),
        ),
        out_shape=jax.ShapeDtypeStruct((ROWS, D_MODEL), f32),
    )(be, valid, xs, w1, b1r, w3, b3r, w2, b2r)

    g1 = ys[p1f]
    g2 = ys[p2f]

    out = pl.pallas_call(
        _combine_kernel,
        out_shape=jax.ShapeDtypeStruct((N_TOKENS, D_MODEL), f32),
    )(g1, g2, p1c, p2c)
    return out


# speedup vs baseline: 3.8538x; 1.2386x over previous
import functools

import jax
import jax.numpy as jnp
from jax import lax
from jax.experimental import pallas as pl
from jax.experimental.pallas import tpu as pltpu
from jax.experimental.pallas import tpu_sc as plsc

D_MODEL = 768
D_FF = 2048
N_EXPERTS = 8
N_TOKENS = 2048
LANES = 128
BR = 256
LOG2_BR = 8
ROWS = N_TOKENS * 2 + N_EXPERTS * BR
NBLK = ROWS // BR


def _gate_kernel(x_ref, gw_ref, gb_ref, w_ref, pos1_ref, pos2_ref,
                 p1_ref, p2_ref, pc_ref):
    l = jnp.dot(x_ref[:], gw_ref[:], preferred_element_type=jnp.float32)
    l = l + gb_ref[:]
    col = jax.lax.broadcasted_iota(jnp.int32, l.shape, 1)
    neg = jnp.float32(-1e30)
    l = jnp.where(col < N_EXPERTS, l, neg)
    m1 = jnp.max(l, axis=1, keepdims=True)
    i1 = jnp.min(jnp.where(l >= m1, col, LANES), axis=1, keepdims=True)
    s = jnp.sum(jnp.exp(l - m1), axis=1, keepdims=True)
    l2 = jnp.where(col == i1, neg, l)
    m2 = jnp.max(l2, axis=1, keepdims=True)
    i2 = jnp.min(jnp.where(l2 >= m2, col, LANES), axis=1, keepdims=True)
    p1 = 1.0 / s
    p2 = jnp.exp(m2 - m1) / s
    d = p1 + p2 + 1e-6
    p1_ref[:] = p1 / d
    p2_ref[:] = p2 / d
    oh1 = jnp.where(col == i1, 1.0, 0.0)
    oh2 = jnp.where(col == i2, 1.0, 0.0)
    w_ref[:] = oh1 * (p1 / d) + oh2 * (p2 / d)
    tot = oh1 + oh2
    r_t = jax.lax.broadcasted_iota(jnp.int32, (N_TOKENS, N_TOKENS), 0)
    c_t = jax.lax.broadcasted_iota(jnp.int32, (N_TOKENS, N_TOKENS), 1)
    lt = jnp.where(r_t > c_t, 1.0, 0.0).astype(jnp.bfloat16)
    excl = jnp.dot(lt, tot.astype(jnp.bfloat16),
                   preferred_element_type=jnp.float32)
    counts = jnp.sum(tot, axis=0, keepdims=True)
    pci = counts.astype(jnp.int32)
    pc = ((pci + (BR - 1)) >> LOG2_BR) << LOG2_BR
    pc_ref[:] = pc
    r_e = jax.lax.broadcasted_iota(jnp.int32, (LANES, LANES), 0)
    c_e = jax.lax.broadcasted_iota(jnp.int32, (LANES, LANES), 1)
    lte = jnp.where(r_e < c_e, 1.0, 0.0)
    seg = jnp.dot(pc.astype(jnp.float32), lte,
                  preferred_element_type=jnp.float32)
    segex = excl + seg
    pos1_ref[:] = jnp.sum(jnp.where(col == i1, segex, 0.0), axis=1,
                          keepdims=True).astype(jnp.int32)
    pos2_ref[:] = jnp.sum(jnp.where(col == i2, segex, 0.0), axis=1,
                          keepdims=True).astype(jnp.int32)


def _ff_sparse_kernel(be_ref, valid_ref, xs_ref, w1_ref, b1_ref, w3_ref,
                      b3_ref, w2_ref, b2_ref, ys_ref):
    g = pl.program_id(0)

    @pl.when(valid_ref[g] == 1)
    def _():
        xb = xs_ref[:]
        h1 = jnp.dot(xb, w1_ref[0], preferred_element_type=jnp.float32)
        h1 = h1 + b1_ref[0]
        h3 = jnp.dot(xb, w3_ref[0], preferred_element_type=jnp.float32)
        h3 = h3 + b3_ref[0]
        h = h1 * jax.nn.sigmoid(h1) * h3
        ys_ref[:] = jnp.dot(h, w2_ref[0],
                            preferred_element_type=jnp.float32) + b2_ref[0]


def _combine_kernel(g1_ref, g2_ref, p1_ref, p2_ref, out_ref):
    out_ref[:] = g1_ref[:] * p1_ref[:] + g2_ref[:] * p2_ref[:]


_SC_INFO = plsc.get_sparse_core_info()
_NC = _SC_INFO.num_cores
_NS = _SC_INFO.num_subcores
_NW = _NC * _NS
_TPW = N_TOKENS // _NW
_SC_MESH = plsc.VectorSubcoreMesh(core_axis_name="c", subcore_axis_name="s")


@functools.partial(
    pl.kernel,
    mesh=_SC_MESH,
    out_type=jax.ShapeDtypeStruct((ROWS, D_MODEL), jnp.float32),
    scratch_types=[
        pltpu.VMEM((_TPW,), jnp.int32),
        pltpu.VMEM((_TPW,), jnp.int32),
        pltpu.VMEM((_TPW, D_MODEL), jnp.float32),
        pltpu.SemaphoreType.DMA,
    ],
)
def _sc_scatter(x_hbm, pos1_hbm, pos2_hbm, xs_hbm, idx1_v, idx2_v, rows_v,
                sem):
    wid = lax.axis_index("s") * _NC + lax.axis_index("c")
    base = wid * _TPW
    pltpu.sync_copy(pos1_hbm.at[pl.ds(base, _TPW)], idx1_v)
    pltpu.sync_copy(pos2_hbm.at[pl.ds(base, _TPW)], idx2_v)
    pltpu.sync_copy(x_hbm.at[pl.ds(base, _TPW)], rows_v)
    pltpu.async_copy(rows_v, xs_hbm.at[idx1_v], sem).wait()
    pltpu.async_copy(rows_v, xs_hbm.at[idx2_v], sem).wait()


@functools.partial(
    pl.kernel,
    mesh=_SC_MESH,
    out_type=(
        jax.ShapeDtypeStruct((N_TOKENS, D_MODEL), jnp.float32),
        jax.ShapeDtypeStruct((N_TOKENS, D_MODEL), jnp.float32),
    ),
    scratch_types=[
        pltpu.VMEM((_TPW,), jnp.int32),
        pltpu.VMEM((_TPW, D_MODEL), jnp.float32),
        pltpu.SemaphoreType.DMA,
    ],
)
def _sc_gather(ys_hbm, pos1_hbm, pos2_hbm, g1_hbm, g2_hbm, idx_v, rows_v,
               sem):
    wid = lax.axis_index("s") * _NC + lax.axis_index("c")
    base = wid * _TPW
    pltpu.sync_copy(pos1_hbm.at[pl.ds(base, _TPW)], idx_v)
    pltpu.async_copy(ys_hbm.at[idx_v], rows_v, sem).wait()
    pltpu.sync_copy(rows_v, g1_hbm.at[pl.ds(base, _TPW)])
    pltpu.sync_copy(pos2_hbm.at[pl.ds(base, _TPW)], idx_v)
    pltpu.async_copy(ys_hbm.at[idx_v], rows_v, sem).wait()
    pltpu.sync_copy(rows_v, g2_hbm.at[pl.ds(base, _TPW)])


@functools.partial(jax.jit, static_argnames=())
def kernel(x, gate_W, gate_b, w1, b1, w2, b2, w3, b3):
    f32 = jnp.float32
    gwp = jnp.pad(gate_W, ((0, 0), (0, LANES - N_EXPERTS)))
    gbp = jnp.pad(gate_b, (0, LANES - N_EXPERTS)).reshape(1, LANES)
    w_te, pos1, pos2, p1c, p2c, pc_row = pl.pallas_call(
        _gate_kernel,
        out_shape=(
            jax.ShapeDtypeStruct((N_TOKENS, LANES), f32),
            jax.ShapeDtypeStruct((N_TOKENS, 1), jnp.int32),
            jax.ShapeDtypeStruct((N_TOKENS, 1), jnp.int32),
            jax.ShapeDtypeStruct((N_TOKENS, 1), f32),
            jax.ShapeDtypeStruct((N_TOKENS, 1), f32),
            jax.ShapeDtypeStruct((1, LANES), jnp.int32),
        ),
    )(x, gwp, gbp)

    pc8 = pc_row[0, :N_EXPERTS]
    ends = jnp.cumsum(pc8)
    gbase = jnp.arange(NBLK, dtype=jnp.int32) * BR
    be = jnp.sum((ends[None, :] <= gbase[:, None]).astype(jnp.int32), axis=1)
    valid = (gbase < ends[-1]).astype(jnp.int32)
    max_e = jnp.max(jnp.where(pc8 > 0, jnp.arange(N_EXPERTS, dtype=jnp.int32),
                              0))
    be = jnp.minimum(be, max_e)

    p1f = pos1.reshape(N_TOKENS)
    p2f = pos2.reshape(N_TOKENS)
    xs = _sc_scatter(x, p1f, p2f)

    b1r = b1.reshape(N_EXPERTS, 1, D_FF)
    b3r = b3.reshape(N_EXPERTS, 1, D_FF)
    b2r = b2.reshape(N_EXPERTS, 1, D_MODEL)
    ys = pl.pallas_call(
        _ff_sparse_kernel,
        grid_spec=pltpu.PrefetchScalarGridSpec(
            num_scalar_prefetch=2,
            grid=(NBLK,),
            in_specs=[
                pl.BlockSpec((BR, D_MODEL), lambda g, be, va: (g, 0)),
                pl.BlockSpec((1, D_MODEL, D_FF), lambda g, be, va: (be[g], 0, 0)),
                pl.BlockSpec((1, 1, D_FF), lambda g, be, va: (be[g], 0, 0)),
                pl.BlockSpec((1, D_MODEL, D_FF), lambda g, be, va: (be[g], 0, 0)),
                pl.BlockSpec((1, 1, D_FF), lambda g, be, va: (be[g], 0, 0)),
                pl.BlockSpec((1, D_FF, D_MODEL), lambda g, be, va: (be[g], 0, 0)),
                pl.BlockSpec((1, 1, D_MODEL), lambda g, be, va: (be[g], 0, 0)),
            ],
            out_specs=pl.BlockSpec((BR, D_MODEL), lambda g, be, va: (g, 0)),
        ),
        out_shape=jax.ShapeDtypeStruct((ROWS, D_MODEL), f32),
    )(be, valid, xs, w1, b1r, w3, b3r, w2, b2r)

    g1, g2 = _sc_gather(ys, p1f, p2f)

    out = pl.pallas_call(
        _combine_kernel,
        out_shape=jax.ShapeDtypeStruct((N_TOKENS, D_MODEL), f32),
    )(g1, g2, p1c, p2c)
    return out
